# baseline (speedup 1.0000x reference)
import functools

import jax
import jax.numpy as jnp
from jax.experimental import pallas as pl
from jax.experimental.pallas import tpu as pltpu

_LANES = 128


def _round_up(a: int, m: int) -> int:
    return (a + m - 1) // m * m


def _cdiv(a: int, m: int) -> int:
    return (a + m - 1) // m


def _fused_body(x_ref, out_ref, *, inv_len):
    x = x_ref[...].astype(jnp.float32)
    out_ref[:, 0] = jnp.max(x, axis=2, keepdims=True).astype(out_ref.dtype)
    out_ref[:, 1] = (jnp.sum(x, axis=2, keepdims=True)
                     * inv_len).astype(out_ref.dtype)


def _pool_body(x_ref, max_ref, avg_ref, *, inv_len):
    x = x_ref[...].astype(jnp.float32)
    max_ref[...] = jnp.max(x, axis=1, keepdims=True).astype(max_ref.dtype)
    avg_ref[...] = (jnp.sum(x, axis=1, keepdims=True)
                    * inv_len).astype(avg_ref.dtype)


def _concat_pool_fused(x, *, batches_per_block):
    N, C, L = x.shape
    nb = batches_per_block
    body = functools.partial(_fused_body, inv_len=1.0 / L)
    out = pl.pallas_call(
        body,
        out_shape=jax.ShapeDtypeStruct((N, 2, C, 1), x.dtype),
        grid=(_cdiv(N, nb),),
        in_specs=[pl.BlockSpec((nb, C, L), lambda i: (i, 0, 0))],
        out_specs=pl.BlockSpec((nb, 2, C, 1), lambda i: (i, 0, 0, 0)),
        compiler_params=pltpu.CompilerParams(
            dimension_semantics=("parallel",)),
    )(x)
    return out.reshape(N, 2 * C, 1)


def _concat_pool_rows(x, *, target_block_bytes=8 * 1024 * 1024):
    N, C, L = x.shape
    NR = N * C
    x2 = x.reshape(NR, L)

    sub = {4: 8, 2: 16, 1: 32}.get(jnp.dtype(x.dtype).itemsize, 8)
    row_bytes = L * jnp.dtype(x.dtype).itemsize
    br = max(sub, _round_up(max(1, target_block_bytes // row_bytes), sub))
    if NR > sub:
        br = min(br, _round_up(_cdiv(NR, 2), sub))
    br = min(br, _round_up(NR, sub))
    nr_blocks = _cdiv(NR, br)

    body = functools.partial(_pool_body, inv_len=1.0 / L)
    mx2, av2 = pl.pallas_call(
        body,
        out_shape=(jax.ShapeDtypeStruct((NR, 1), x.dtype),
                   jax.ShapeDtypeStruct((NR, 1), x.dtype)),
        grid=(nr_blocks,),
        in_specs=[pl.BlockSpec((br, L), lambda i: (i, 0))],
        out_specs=[pl.BlockSpec((br, 1), lambda i: (i, 0)),
                   pl.BlockSpec((br, 1), lambda i: (i, 0))],
        compiler_params=pltpu.CompilerParams(
            dimension_semantics=("parallel",)),
    )(x2)

    mx = mx2.reshape(N, C)
    av = av2.reshape(N, C)
    return jnp.concatenate([mx, av], axis=1)[:, :, None]


def kernel(x):
    N, C, L = x.shape
    block_bytes = C * L * jnp.dtype(x.dtype).itemsize
    if C % 8 == 0 and N >= 2 and block_bytes <= 16 * 1024 * 1024:
        nb = max(1, (8 * 1024 * 1024) // block_bytes)
        while nb > 1 and _cdiv(N, nb) < 2:
            nb //= 2
        return _concat_pool_fused(x, batches_per_block=nb)
    return _concat_pool_rows(x)


# speedup vs baseline: 1.0249x; 1.0249x over previous
import functools

import jax
import jax.numpy as jnp
from jax.experimental import pallas as pl
from jax.experimental.pallas import tpu as pltpu

_LANES = 128


def _round_up(a: int, m: int) -> int:
    return (a + m - 1) // m * m


def _cdiv(a: int, m: int) -> int:
    return (a + m - 1) // m


def _fused_body(x_ref, out_ref, *, inv_len):
    x = x_ref[...].astype(jnp.float32)
    out_ref[:, 0] = jnp.max(x, axis=2, keepdims=True).astype(out_ref.dtype)
    out_ref[:, 1] = (jnp.sum(x, axis=2, keepdims=True)
                     * inv_len).astype(out_ref.dtype)


def _pool_body(x_ref, max_ref, avg_ref, *, inv_len):
    x = x_ref[...].astype(jnp.float32)
    max_ref[...] = jnp.max(x, axis=1, keepdims=True).astype(max_ref.dtype)
    avg_ref[...] = (jnp.sum(x, axis=1, keepdims=True)
                    * inv_len).astype(avg_ref.dtype)


def _concat_pool_fused(x, *, batches_per_block):
    N, C, L = x.shape
    nb = batches_per_block
    body = functools.partial(_fused_body, inv_len=1.0 / L)
    out = pl.pallas_call(
        body,
        out_shape=jax.ShapeDtypeStruct((N, 2, C, 1), x.dtype),
        grid=(_cdiv(N, nb),),
        in_specs=[pl.BlockSpec((nb, C, L), lambda i: (i, 0, 0))],
        out_specs=pl.BlockSpec((nb, 2, C, 1), lambda i: (i, 0, 0, 0)),
        compiler_params=pltpu.CompilerParams(
            dimension_semantics=("parallel",)),
    )(x)
    return out.reshape(N, 2 * C, 1)


def _concat_pool_rows(x, *, target_block_bytes=8 * 1024 * 1024):
    N, C, L = x.shape
    NR = N * C
    x2 = x.reshape(NR, L)

    sub = {4: 8, 2: 16, 1: 32}.get(jnp.dtype(x.dtype).itemsize, 8)
    row_bytes = L * jnp.dtype(x.dtype).itemsize
    br = max(sub, _round_up(max(1, target_block_bytes // row_bytes), sub))
    if NR > sub:
        br = min(br, _round_up(_cdiv(NR, 2), sub))
    br = min(br, _round_up(NR, sub))
    nr_blocks = _cdiv(NR, br)

    body = functools.partial(_pool_body, inv_len=1.0 / L)
    mx2, av2 = pl.pallas_call(
        body,
        out_shape=(jax.ShapeDtypeStruct((NR, 1), x.dtype),
                   jax.ShapeDtypeStruct((NR, 1), x.dtype)),
        grid=(nr_blocks,),
        in_specs=[pl.BlockSpec((br, L), lambda i: (i, 0))],
        out_specs=[pl.BlockSpec((br, 1), lambda i: (i, 0)),
                   pl.BlockSpec((br, 1), lambda i: (i, 0))],
        compiler_params=pltpu.CompilerParams(
            dimension_semantics=("parallel",)),
    )(x2)

    mx = mx2.reshape(N, C)
    av = av2.reshape(N, C)
    return jnp.concatenate([mx, av], axis=1)[:, :, None]


def kernel(x):
    N, C, L = x.shape
    block_bytes = C * L * jnp.dtype(x.dtype).itemsize
    if C % 8 == 0 and N >= 2 and block_bytes <= 16 * 1024 * 1024:
        nb = max(1, (20 * 1024 * 1024) // block_bytes)
        while nb > 1 and _cdiv(N, nb) < 2:
            nb //= 2
        return _concat_pool_fused(x, batches_per_block=nb)
    return _concat_pool_rows(x)


# speedup vs baseline: 1.0286x; 1.0036x over previous
import functools

import jax
import jax.numpy as jnp
from jax.experimental import pallas as pl
from jax.experimental.pallas import tpu as pltpu

_LANES = 128


def _round_up(a: int, m: int) -> int:
    return (a + m - 1) // m * m


def _cdiv(a: int, m: int) -> int:
    return (a + m - 1) // m


def _fused_body(x_ref, out_ref, *, inv_len):
    x = x_ref[...].astype(jnp.float32)
    out_ref[:, 0] = jnp.max(x, axis=2, keepdims=True).astype(out_ref.dtype)
    out_ref[:, 1] = (jnp.sum(x, axis=2, keepdims=True)
                     * inv_len).astype(out_ref.dtype)


def _pool_body(x_ref, max_ref, avg_ref, *, inv_len):
    x = x_ref[...].astype(jnp.float32)
    max_ref[...] = jnp.max(x, axis=1, keepdims=True).astype(max_ref.dtype)
    avg_ref[...] = (jnp.sum(x, axis=1, keepdims=True)
                    * inv_len).astype(avg_ref.dtype)


def _concat_pool_fused(x, *, batches_per_block):
    N, C, L = x.shape
    nb = batches_per_block
    body = functools.partial(_fused_body, inv_len=1.0 / L)
    out = pl.pallas_call(
        body,
        out_shape=jax.ShapeDtypeStruct((N, 2, C, 1), x.dtype),
        grid=(_cdiv(N, nb),),
        in_specs=[pl.BlockSpec((nb, C, L), lambda i: (i, 0, 0))],
        out_specs=pl.BlockSpec((nb, 2, C, 1), lambda i: (i, 0, 0, 0)),
        compiler_params=pltpu.CompilerParams(
            dimension_semantics=("parallel",)),
    )(x)
    return out.reshape(N, 2 * C, 1)


def _concat_pool_rows(x, *, target_block_bytes=8 * 1024 * 1024):
    N, C, L = x.shape
    NR = N * C
    x2 = x.reshape(NR, L)

    sub = {4: 8, 2: 16, 1: 32}.get(jnp.dtype(x.dtype).itemsize, 8)
    row_bytes = L * jnp.dtype(x.dtype).itemsize
    br = max(sub, _round_up(max(1, target_block_bytes // row_bytes), sub))
    if NR > sub:
        br = min(br, _round_up(_cdiv(NR, 2), sub))
    br = min(br, _round_up(NR, sub))
    nr_blocks = _cdiv(NR, br)

    body = functools.partial(_pool_body, inv_len=1.0 / L)
    mx2, av2 = pl.pallas_call(
        body,
        out_shape=(jax.ShapeDtypeStruct((NR, 1), x.dtype),
                   jax.ShapeDtypeStruct((NR, 1), x.dtype)),
        grid=(nr_blocks,),
        in_specs=[pl.BlockSpec((br, L), lambda i: (i, 0))],
        out_specs=[pl.BlockSpec((br, 1), lambda i: (i, 0)),
                   pl.BlockSpec((br, 1), lambda i: (i, 0))],
        compiler_params=pltpu.CompilerParams(
            dimension_semantics=("parallel",)),
    )(x2)

    mx = mx2.reshape(N, C)
    av = av2.reshape(N, C)
    return jnp.concatenate([mx, av], axis=1)[:, :, None]


def kernel(x):
    N, C, L = x.shape
    block_bytes = C * L * jnp.dtype(x.dtype).itemsize
    if False and C % 8 == 0 and N >= 2 and block_bytes <= 16 * 1024 * 1024:
        nb = max(1, (20 * 1024 * 1024) // block_bytes)
        while nb > 1 and _cdiv(N, nb) < 2:
            nb //= 2
        return _concat_pool_fused(x, batches_per_block=nb)
    return _concat_pool_rows(x, target_block_bytes=10 * 1024 * 1024)
